# Initial kernel scaffold; baseline (speedup 1.0000x reference)
import jax
import jax.numpy as jnp
from jax.experimental import pallas as pl


def kernel(x, edge_index, W1, a_src1, a_dst1, W2, a_src2, a_dst2):
    raise NotImplementedError("write your pallas kernel here")


# speedup vs baseline: 58.8058x; 58.8058x over previous
import jax
import jax.numpy as jnp
from jax import lax
from jax.experimental import pallas as pl
from jax.experimental.pallas import tpu as pltpu
from jax.experimental.pallas import tpu_sc as plsc

N = 10000
NP = 10240
D = 128
HF = 64
E = 320000
E1 = E + N
NC, NS = 2, 16
NW = NC * NS
CH = 128
NCHK = 81
EPW = NCHK * CH
EPAD = EPW * NW
RPS = NP // NS
BLK = 1024
BIG = -1e30

_mesh = plsc.VectorSubcoreMesh(core_axis_name="c", subcore_axis_name="s",
                               num_cores=NC, num_subcores=NS)


_SC_PARAMS = pltpu.CompilerParams(use_tc_tiling_on_sc=False,
                                  needs_layout_passes=False)


def _logits_body(ta_hbm, tb_hbm, src_hbm, dst_hbm, zn_hbm,
                 ex_hbm, s_hbm,
                 src_c, dst_c, ra, rb, exc, sacc):
    c = lax.axis_index("c")
    sid = lax.axis_index("s")
    wid = sid * NC + c
    r0 = sid * RPS
    pltpu.sync_copy(zn_hbm.at[pl.ds(r0, RPS)], sacc.at[pl.ds(r0, RPS)])
    plsc.subcore_barrier()
    base = wid * EPW

    @pl.loop(0, NCHK)
    def _chunk(i):
        off = pl.multiple_of(base + i * CH, CH)
        pltpu.sync_copy(src_hbm.at[pl.ds(off, CH)], src_c)
        pltpu.sync_copy(dst_hbm.at[pl.ds(off, CH)], dst_c)
        pltpu.sync_copy(ta_hbm.at[src_c], ra)
        pltpu.sync_copy(tb_hbm.at[dst_c], rb)
        for r in range(CH):
            v = ra[r, :] + rb[r, :]
            v = jnp.where(v > 0, v, 0.2 * v)
            exc[r, :] = jnp.exp(v)
        pltpu.sync_copy(exc, ex_hbm.at[pl.ds(off, CH)])
        pltpu.sync_copy(exc, sacc.at[dst_c], add=True)

    plsc.subcore_barrier()
    pltpu.sync_copy(sacc.at[pl.ds(r0, RPS)], s_hbm.at[c, pl.ds(r0, RPS)])


_sc_logits = pl.kernel(
    _logits_body,
    out_type=(jax.ShapeDtypeStruct((EPAD, 16), jnp.float32),
              jax.ShapeDtypeStruct((NC, NP, 16), jnp.float32)),
    mesh=_mesh,
    scratch_types=[pltpu.VMEM((CH,), jnp.int32),
                   pltpu.VMEM((CH,), jnp.int32),
                   pltpu.VMEM((CH, 16), jnp.float32),
                   pltpu.VMEM((CH, 16), jnp.float32),
                   pltpu.VMEM((CH, 16), jnp.float32),
                   pltpu.VMEM_SHARED((NP, 16), jnp.float32)],
    compiler_params=_SC_PARAMS,
)


def _make_sc_agg(wf, head_stride):
    nb = wf // 16

    def body(ex_hbm, s0_hbm, s1_hbm, wh_hbm, src_hbm, dst_hbm, zn_hbm,
             out_hbm,
             src_c, dst_c, exc, s0b, s1b, whr, msg, oacc):
        c = lax.axis_index("c")
        sid = lax.axis_index("s")
        wid = sid * NC + c
        r0 = sid * RPS
        pltpu.sync_copy(zn_hbm.at[pl.ds(r0, RPS)], oacc.at[pl.ds(r0, RPS)])
        plsc.subcore_barrier()
        base = wid * EPW

        @pl.loop(0, NCHK)
        def _chunk(i):
            off = pl.multiple_of(base + i * CH, CH)
            pltpu.sync_copy(src_hbm.at[pl.ds(off, CH)], src_c)
            pltpu.sync_copy(dst_hbm.at[pl.ds(off, CH)], dst_c)
            pltpu.sync_copy(ex_hbm.at[pl.ds(off, CH)], exc)
            pltpu.sync_copy(s0_hbm.at[dst_c], s0b)
            pltpu.sync_copy(s1_hbm.at[dst_c], s1b)
            pltpu.sync_copy(wh_hbm.at[src_c], whr)
            lane = lax.iota(jnp.int32, 16)
            zero16 = lane & 0
            lane_h = (lane >> 3) * head_stride
            for r in range(CH):
                attn = exc[r, :] / (s0b[r, :] + s1b[r, :] + 1e-30)
                if head_stride == 0:
                    a0 = jnp.sum(jnp.where(lane == 0, attn, 0.0))
                    ab = jnp.broadcast_to(a0, (16,))
                    msg[r, :] = ab * whr[r, :]
                else:
                    exc[r, :] = attn
                    for j in range(nb):
                        ab = plsc.load_gather(
                            exc, [zero16 + r, lane_h + 2 * j * head_stride])
                        msg[r, pl.ds(16 * j, 16)] = ab * whr[r, pl.ds(16 * j, 16)]
            pltpu.sync_copy(msg, oacc.at[dst_c], add=True)

        plsc.subcore_barrier()
        pltpu.sync_copy(oacc.at[pl.ds(r0, RPS)], out_hbm.at[c, pl.ds(r0, RPS)])

    return pl.kernel(
        body,
        out_type=jax.ShapeDtypeStruct((NC, NP, wf), jnp.float32),
        mesh=_mesh,
        scratch_types=[pltpu.VMEM((CH,), jnp.int32),
                       pltpu.VMEM((CH,), jnp.int32),
                       pltpu.VMEM((CH, 16), jnp.float32),
                       pltpu.VMEM((CH, 16), jnp.float32),
                       pltpu.VMEM((CH, 16), jnp.float32),
                       pltpu.VMEM((CH, wf), jnp.float32),
                       pltpu.VMEM((CH, wf), jnp.float32),
                       pltpu.VMEM_SHARED((NP, wf), jnp.float32)],
        compiler_params=_SC_PARAMS,
    )


_sc_agg64 = _make_sc_agg(HF, 1)
_sc_agg16 = _make_sc_agg(16, 0)


def _proj1_body(x_ref, w_ref, wh_ref, ta_ref, tb_ref):
    y = jnp.dot(x_ref[...], w_ref[...], preferred_element_type=jnp.float32)
    wh_ref[...] = y[:, :HF]
    rows = pl.program_id(0) * BLK + lax.broadcasted_iota(jnp.int32, (BLK, 16), 0)
    valid = rows < N
    ta_ref[...] = jnp.where(valid, y[:, HF:HF + 16], BIG)
    tb_ref[...] = jnp.where(valid, y[:, HF + 16:HF + 32], BIG)


_proj1 = pl.pallas_call(
    _proj1_body,
    grid=(NP // BLK,),
    in_specs=[pl.BlockSpec((BLK, D), lambda i: (i, 0)),
              pl.BlockSpec((D, D), lambda i: (0, 0))],
    out_specs=[pl.BlockSpec((BLK, HF), lambda i: (i, 0)),
               pl.BlockSpec((BLK, 16), lambda i: (i, 0)),
               pl.BlockSpec((BLK, 16), lambda i: (i, 0))],
    out_shape=[jax.ShapeDtypeStruct((NP, HF), jnp.float32),
               jax.ShapeDtypeStruct((NP, 16), jnp.float32),
               jax.ShapeDtypeStruct((NP, 16), jnp.float32)],
)


def _proj2_body(oa_ref, ob_ref, w_ref, wh_ref, ta_ref, tb_ref):
    hsum = oa_ref[...] + ob_ref[...]
    h = jnp.where(hsum > 0, hsum, jnp.exp(hsum) - 1.0)
    y = jnp.dot(h, w_ref[...], preferred_element_type=jnp.float32)
    col = lax.broadcasted_iota(jnp.int32, (BLK, 16), 1)
    rows = pl.program_id(0) * BLK + lax.broadcasted_iota(jnp.int32, (BLK, 16), 0)
    valid = rows < N
    wh_ref[...] = jnp.where(col < 7, y, 0.0)
    asl = y[:, 7:8]
    adl = y[:, 8:9]
    ta = jnp.where(col == 0, asl, 0.0) + jnp.where(col == 8, adl, 0.0)
    tb = jnp.where(col == 0, adl, 0.0) + jnp.where(col == 8, asl, 0.0)
    ta_ref[...] = jnp.where(valid, ta, BIG)
    tb_ref[...] = jnp.where(valid, tb, BIG)


_proj2 = pl.pallas_call(
    _proj2_body,
    grid=(NP // BLK,),
    in_specs=[pl.BlockSpec((BLK, HF), lambda i: (i, 0)),
              pl.BlockSpec((BLK, HF), lambda i: (i, 0)),
              pl.BlockSpec((HF, 16), lambda i: (0, 0))],
    out_specs=[pl.BlockSpec((BLK, 16), lambda i: (i, 0)),
               pl.BlockSpec((BLK, 16), lambda i: (i, 0)),
               pl.BlockSpec((BLK, 16), lambda i: (i, 0))],
    out_shape=[jax.ShapeDtypeStruct((NP, 16), jnp.float32),
               jax.ShapeDtypeStruct((NP, 16), jnp.float32),
               jax.ShapeDtypeStruct((NP, 16), jnp.float32)],
)


def _smax_body(oa_ref, ob_ref, o_ref):
    z = oa_ref[...] + ob_ref[...]
    col = lax.broadcasted_iota(jnp.int32, z.shape, 1)
    zm = jnp.where(col < 7, z, BIG)
    m = jnp.max(zm, axis=1, keepdims=True)
    e = jnp.exp(zm - m)
    o_ref[...] = e / jnp.sum(e, axis=1, keepdims=True)


_smax = pl.pallas_call(
    _smax_body,
    grid=(NP // BLK,),
    in_specs=[pl.BlockSpec((BLK, 16), lambda i: (i, 0)),
              pl.BlockSpec((BLK, 16), lambda i: (i, 0))],
    out_specs=pl.BlockSpec((BLK, 16), lambda i: (i, 0)),
    out_shape=jax.ShapeDtypeStruct((NP, 16), jnp.float32),
)


@jax.jit
def kernel(x, edge_index, W1, a_src1, a_dst1, W2, a_src2, a_dst2):
    f32 = jnp.float32
    W1r = jnp.transpose(W1, (1, 0, 2)).reshape(D, HF)
    rows64 = jnp.arange(HF)
    heads = rows64 // 8
    As = jnp.zeros((HF, 8), f32).at[rows64, heads].set(a_src1.reshape(-1))
    Ad = jnp.zeros((HF, 8), f32).at[rows64, heads].set(a_dst1.reshape(-1))
    Wc1 = jnp.concatenate(
        [W1r, W1r @ As, W1r @ Ad, W1r @ Ad, W1r @ As,
         jnp.zeros((D, D - HF - 32), f32)], axis=1)
    W2r = W2[0]
    as2 = a_src2.reshape(-1)
    ad2 = a_dst2.reshape(-1)
    Wc2 = jnp.concatenate(
        [W2r, (W2r @ as2)[:, None], (W2r @ ad2)[:, None],
         jnp.zeros((HF, 7), f32)], axis=1)
    xp = jnp.pad(x, ((0, NP - N), (0, 0)))
    loops = jnp.arange(N, dtype=jnp.int32)
    padi = jnp.full((EPAD - E1,), N, jnp.int32)
    src = jnp.concatenate([edge_index[0], loops, padi])
    dst = jnp.concatenate([edge_index[1], loops, padi])
    zn16 = jnp.zeros((NP, 16), f32)
    zn64 = jnp.zeros((NP, HF), f32)

    wh1, t1a, t1b = _proj1(xp, Wc1)
    ex1, s1 = _sc_logits(t1a, t1b, src, dst, zn16)
    o1 = _sc_agg64(ex1, s1[0], s1[1], wh1, src, dst, zn64)
    wh2, t2a, t2b = _proj2(o1[0], o1[1], Wc2)
    ex2, s2 = _sc_logits(t2a, t2b, src, dst, zn16)
    o2 = _sc_agg16(ex2, s2[0], s2[1], wh2, src, dst, zn16)
    probs = _smax(o2[0], o2[1])
    return probs[:N, :7]


# speedup vs baseline: 89.3500x; 1.5194x over previous
import jax
import jax.numpy as jnp
from jax import lax
from jax.experimental import pallas as pl
from jax.experimental.pallas import tpu as pltpu
from jax.experimental.pallas import tpu_sc as plsc

N = 10000
NP = 10240
D = 128
HF = 64
E = 320000
E1 = E + N
NC, NS = 2, 16
NW = NC * NS
CH = 128
NCHK = 81
EPW = NCHK * CH
EPAD = EPW * NW
RPS = NP // NS
BLK = 1024
BIG = -1e30

_mesh = plsc.VectorSubcoreMesh(core_axis_name="c", subcore_axis_name="s",
                               num_cores=NC, num_subcores=NS)


_SC_PARAMS = pltpu.CompilerParams(use_tc_tiling_on_sc=False,
                                  needs_layout_passes=False)


def _make_sc_layer(wf, multi_head):
    nb = wf // 16

    def body(ta_hbm, tb_hbm, wh_hbm, src_hbm, dst_hbm, zn16_hbm, znw_hbm,
             s_hbm, out_hbm,
             src_c, dst_c, ra, rb, exc, whr, msg, sacc, oacc):
        c = lax.axis_index("c")
        sid = lax.axis_index("s")
        wid = sid * NC + c
        r0 = sid * RPS
        pltpu.sync_copy(zn16_hbm.at[pl.ds(r0, RPS)], sacc.at[pl.ds(r0, RPS)])
        pltpu.sync_copy(znw_hbm.at[pl.ds(r0, RPS)], oacc.at[pl.ds(r0, RPS)])
        plsc.subcore_barrier()
        base = wid * EPW

        @pl.loop(0, NCHK)
        def _chunk(i):
            off = pl.multiple_of(base + i * CH, CH)
            pltpu.sync_copy(src_hbm.at[pl.ds(off, CH)], src_c)
            pltpu.sync_copy(dst_hbm.at[pl.ds(off, CH)], dst_c)
            pltpu.sync_copy(ta_hbm.at[src_c], ra)
            pltpu.sync_copy(tb_hbm.at[dst_c], rb)
            pltpu.sync_copy(wh_hbm.at[src_c], whr)
            lane = lax.iota(jnp.int32, 16)
            zero16 = lane & 0
            lane_h = lane >> 3
            for r in range(CH):
                v = ra[r, :] + rb[r, :]
                v = jnp.where(v > 0, v, 0.2 * v)
                exv = jnp.exp(v)
                exc[r, :] = exv
                if multi_head:
                    for j in range(nb):
                        ab = plsc.load_gather(exc, [zero16 + r, lane_h + 2 * j])
                        msg[r, pl.ds(16 * j, 16)] = ab * whr[r, pl.ds(16 * j, 16)]
                else:
                    a0 = jnp.sum(jnp.where(lane == 0, exv, 0.0))
                    ab = jnp.broadcast_to(a0, (16,))
                    msg[r, :] = ab * whr[r, :]
            pltpu.sync_copy(exc, sacc.at[dst_c], add=True)
            pltpu.sync_copy(msg, oacc.at[dst_c], add=True)

        plsc.subcore_barrier()
        pltpu.sync_copy(sacc.at[pl.ds(r0, RPS)], s_hbm.at[c, pl.ds(r0, RPS)])
        pltpu.sync_copy(oacc.at[pl.ds(r0, RPS)], out_hbm.at[c, pl.ds(r0, RPS)])

    return pl.kernel(
        body,
        out_type=(jax.ShapeDtypeStruct((NC, NP, 16), jnp.float32),
                  jax.ShapeDtypeStruct((NC, NP, wf), jnp.float32)),
        mesh=_mesh,
        scratch_types=[pltpu.VMEM((CH,), jnp.int32),
                       pltpu.VMEM((CH,), jnp.int32),
                       pltpu.VMEM((CH, 16), jnp.float32),
                       pltpu.VMEM((CH, 16), jnp.float32),
                       pltpu.VMEM((CH, 16), jnp.float32),
                       pltpu.VMEM((CH, wf), jnp.float32),
                       pltpu.VMEM((CH, wf), jnp.float32),
                       pltpu.VMEM_SHARED((NP, 16), jnp.float32),
                       pltpu.VMEM_SHARED((NP, wf), jnp.float32)],
        compiler_params=_SC_PARAMS,
    )


_sc_layer64 = _make_sc_layer(HF, True)
_sc_layer16 = _make_sc_layer(16, False)


def _proj1_body(x_ref, w_ref, wh_ref, ta_ref, tb_ref):
    y = jnp.dot(x_ref[...], w_ref[...], preferred_element_type=jnp.float32)
    wh_ref[...] = y[:, :HF]
    rows = pl.program_id(0) * BLK + lax.broadcasted_iota(jnp.int32, (BLK, 16), 0)
    valid = rows < N
    ta_ref[...] = jnp.where(valid, y[:, HF:HF + 16], BIG)
    tb_ref[...] = jnp.where(valid, y[:, HF + 16:HF + 32], BIG)


_proj1 = pl.pallas_call(
    _proj1_body,
    grid=(NP // BLK,),
    in_specs=[pl.BlockSpec((BLK, D), lambda i: (i, 0)),
              pl.BlockSpec((D, D), lambda i: (0, 0))],
    out_specs=[pl.BlockSpec((BLK, HF), lambda i: (i, 0)),
               pl.BlockSpec((BLK, 16), lambda i: (i, 0)),
               pl.BlockSpec((BLK, 16), lambda i: (i, 0))],
    out_shape=[jax.ShapeDtypeStruct((NP, HF), jnp.float32),
               jax.ShapeDtypeStruct((NP, 16), jnp.float32),
               jax.ShapeDtypeStruct((NP, 16), jnp.float32)],
)


def _proj2_body(oa_ref, ob_ref, sa_ref, sb_ref, w_ref, wh_ref, ta_ref, tb_ref):
    s = sa_ref[...] + sb_ref[...] + 1e-30
    sx = jnp.concatenate(
        [jnp.broadcast_to(s[:, hh:hh + 1], (BLK, 8)) for hh in range(8)],
        axis=1)
    hsum = (oa_ref[...] + ob_ref[...]) / sx
    h = jnp.where(hsum > 0, hsum, jnp.exp(hsum) - 1.0)
    y = jnp.dot(h, w_ref[...], preferred_element_type=jnp.float32)
    col = lax.broadcasted_iota(jnp.int32, (BLK, 16), 1)
    rows = pl.program_id(0) * BLK + lax.broadcasted_iota(jnp.int32, (BLK, 16), 0)
    valid = rows < N
    wh_ref[...] = jnp.where(col < 7, y, 0.0)
    asl = y[:, 7:8]
    adl = y[:, 8:9]
    ta = jnp.where(col == 0, asl, 0.0) + jnp.where(col == 8, adl, 0.0)
    tb = jnp.where(col == 0, adl, 0.0) + jnp.where(col == 8, asl, 0.0)
    ta_ref[...] = jnp.where(valid, ta, BIG)
    tb_ref[...] = jnp.where(valid, tb, BIG)


_proj2 = pl.pallas_call(
    _proj2_body,
    grid=(NP // BLK,),
    in_specs=[pl.BlockSpec((BLK, HF), lambda i: (i, 0)),
              pl.BlockSpec((BLK, HF), lambda i: (i, 0)),
              pl.BlockSpec((BLK, 16), lambda i: (i, 0)),
              pl.BlockSpec((BLK, 16), lambda i: (i, 0)),
              pl.BlockSpec((HF, 16), lambda i: (0, 0))],
    out_specs=[pl.BlockSpec((BLK, 16), lambda i: (i, 0)),
               pl.BlockSpec((BLK, 16), lambda i: (i, 0)),
               pl.BlockSpec((BLK, 16), lambda i: (i, 0))],
    out_shape=[jax.ShapeDtypeStruct((NP, 16), jnp.float32),
               jax.ShapeDtypeStruct((NP, 16), jnp.float32),
               jax.ShapeDtypeStruct((NP, 16), jnp.float32)],
)


def _smax_body(oa_ref, ob_ref, sa_ref, sb_ref, o_ref):
    s = sa_ref[...] + sb_ref[...] + 1e-30
    z = (oa_ref[...] + ob_ref[...]) / s[:, 0:1]
    col = lax.broadcasted_iota(jnp.int32, z.shape, 1)
    zm = jnp.where(col < 7, z, BIG)
    m = jnp.max(zm, axis=1, keepdims=True)
    e = jnp.exp(zm - m)
    o_ref[...] = e / jnp.sum(e, axis=1, keepdims=True)


_smax = pl.pallas_call(
    _smax_body,
    grid=(NP // BLK,),
    in_specs=[pl.BlockSpec((BLK, 16), lambda i: (i, 0)),
              pl.BlockSpec((BLK, 16), lambda i: (i, 0)),
              pl.BlockSpec((BLK, 16), lambda i: (i, 0)),
              pl.BlockSpec((BLK, 16), lambda i: (i, 0))],
    out_specs=pl.BlockSpec((BLK, 16), lambda i: (i, 0)),
    out_shape=jax.ShapeDtypeStruct((NP, 16), jnp.float32),
)


@jax.jit
def kernel(x, edge_index, W1, a_src1, a_dst1, W2, a_src2, a_dst2):
    f32 = jnp.float32
    W1r = jnp.transpose(W1, (1, 0, 2)).reshape(D, HF)
    rows64 = jnp.arange(HF)
    heads = rows64 // 8
    As = jnp.zeros((HF, 8), f32).at[rows64, heads].set(a_src1.reshape(-1))
    Ad = jnp.zeros((HF, 8), f32).at[rows64, heads].set(a_dst1.reshape(-1))
    Wc1 = jnp.concatenate(
        [W1r, W1r @ As, W1r @ Ad, W1r @ Ad, W1r @ As,
         jnp.zeros((D, D - HF - 32), f32)], axis=1)
    W2r = W2[0]
    as2 = a_src2.reshape(-1)
    ad2 = a_dst2.reshape(-1)
    Wc2 = jnp.concatenate(
        [W2r, (W2r @ as2)[:, None], (W2r @ ad2)[:, None],
         jnp.zeros((HF, 7), f32)], axis=1)
    xp = jnp.pad(x, ((0, NP - N), (0, 0)))
    loops = jnp.arange(N, dtype=jnp.int32)
    padi = jnp.full((EPAD - E1,), N, jnp.int32)
    src = jnp.concatenate([edge_index[0], loops, padi])
    dst = jnp.concatenate([edge_index[1], loops, padi])
    zn16 = jnp.zeros((NP, 16), f32)
    zn64 = jnp.zeros((NP, HF), f32)

    wh1, t1a, t1b = _proj1(xp, Wc1)
    s1, o1 = _sc_layer64(t1a, t1b, wh1, src, dst, zn16, zn64)
    wh2, t2a, t2b = _proj2(o1[0], o1[1], s1[0], s1[1], Wc2)
    s2, o2 = _sc_layer16(t2a, t2b, wh2, src, dst, zn16, zn16)
    probs = _smax(o2[0], o2[1], s2[0], s2[1])
    return probs[:N, :7]


# speedup vs baseline: 131.6943x; 1.4739x over previous
import jax
import jax.numpy as jnp
from jax import lax
from jax.experimental import pallas as pl
from jax.experimental.pallas import tpu as pltpu
from jax.experimental.pallas import tpu_sc as plsc

N = 10000
NP = 10240
D = 128
HF = 64
E = 320000
E1 = E + N
NC, NS = 2, 16
NW = NC * NS
CH = 128
NCHK = 82
EPW = NCHK * CH
EPAD = EPW * NW
RPS = NP // NS
BLK = 1024
BIG = -1e30

_mesh = plsc.VectorSubcoreMesh(core_axis_name="c", subcore_axis_name="s",
                               num_cores=NC, num_subcores=NS)


_SC_PARAMS = pltpu.CompilerParams(use_tc_tiling_on_sc=False,
                                  needs_layout_passes=False)


def _make_sc_layer(wf, multi_head):
    nb = wf // 16

    def body(ta_hbm, tb_hbm, wh_hbm, src_hbm, dst_hbm, zn16_hbm, znw_hbm,
             s_hbm, out_hbm,
             src0, src1, dst0, dst1, sdst0, sdst1,
             ra0, ra1, rb0, rb1, whr0, whr1, exc0, exc1, msg0, msg1,
             sacc, oacc, gsem0, gsem1, isem0, isem1, ssem0, ssem1):
        src_c = [src0, src1]
        dst_c = [dst0, dst1]
        sdst = [sdst0, sdst1]
        ra = [ra0, ra1]
        rb = [rb0, rb1]
        whr = [whr0, whr1]
        exc = [exc0, exc1]
        msg = [msg0, msg1]
        gsem = [gsem0, gsem1]
        isem = [isem0, isem1]
        ssem = [ssem0, ssem1]

        c = lax.axis_index("c")
        sid = lax.axis_index("s")
        wid = sid * NC + c
        r0 = sid * RPS
        pltpu.sync_copy(zn16_hbm.at[pl.ds(r0, RPS)], sacc.at[pl.ds(r0, RPS)])
        pltpu.sync_copy(znw_hbm.at[pl.ds(r0, RPS)], oacc.at[pl.ds(r0, RPS)])
        plsc.subcore_barrier()
        base = wid * EPW

        def idx_off(n):
            nn = jnp.minimum(n, NCHK - 1)
            return pl.multiple_of(base + nn * CH, CH)

        def issue_gathers(q):
            pltpu.async_copy(ta_hbm.at[src_c[q]], ra[q], gsem[q])
            pltpu.async_copy(tb_hbm.at[dst_c[q]], rb[q], gsem[q])
            pltpu.async_copy(wh_hbm.at[src_c[q]], whr[q], gsem[q])

        def wait_gathers(q):
            pltpu.make_async_copy(ta_hbm.at[src_c[q]], ra[q], gsem[q]).wait()
            pltpu.make_async_copy(tb_hbm.at[dst_c[q]], rb[q], gsem[q]).wait()
            pltpu.make_async_copy(wh_hbm.at[src_c[q]], whr[q], gsem[q]).wait()

        def issue_idx(n, q, sync=False):
            off = idx_off(n)
            if sync:
                pltpu.sync_copy(src_hbm.at[pl.ds(off, CH)], src_c[q])
                pltpu.sync_copy(dst_hbm.at[pl.ds(off, CH)], dst_c[q])
            else:
                pltpu.async_copy(src_hbm.at[pl.ds(off, CH)], src_c[q], isem[q])
                pltpu.async_copy(dst_hbm.at[pl.ds(off, CH)], dst_c[q], isem[q])

        def wait_idx(n, q):
            off = idx_off(n)
            pltpu.make_async_copy(src_hbm.at[pl.ds(off, CH)], src_c[q], isem[q]).wait()
            pltpu.make_async_copy(dst_hbm.at[pl.ds(off, CH)], dst_c[q], isem[q]).wait()

        def wait_scatters(p):
            pltpu.make_async_copy(exc[p], sacc.at[sdst[p]], ssem[p]).wait()
            pltpu.make_async_copy(msg[p], oacc.at[sdst[p]], ssem[p]).wait()

        issue_idx(0, 0, sync=True)
        issue_idx(1, 1, sync=True)
        issue_gathers(0)

        @pl.loop(0, NCHK, step=2)
        def _pair(i):
            for b in range(2):
                p, q = b, 1 - b
                n = i + b
                @pl.when(n >= 2)
                def _(p=p):
                    wait_scatters(p)
                @pl.when(n >= 1)
                def _(n=n, q=q):
                    wait_idx(n + 1, q)
                issue_gathers(q)
                wait_gathers(p)
                for k in range(CH // 16):
                    sdst[p][pl.ds(16 * k, 16)] = dst_c[p][pl.ds(16 * k, 16)]
                issue_idx(n + 2, p)
                lane = lax.iota(jnp.int32, 16)
                zero16 = lane & 0
                lane_h = lane >> 3
                for r in range(CH):
                    v = ra[p][r, :] + rb[p][r, :]
                    v = jnp.where(v > 0, v, 0.2 * v)
                    exv = jnp.exp(v)
                    exc[p][r, :] = exv
                    if multi_head:
                        for j in range(nb):
                            ab = plsc.load_gather(
                                exc[p], [zero16 + r, lane_h + 2 * j])
                            msg[p][r, pl.ds(16 * j, 16)] = (
                                ab * whr[p][r, pl.ds(16 * j, 16)])
                    else:
                        a0 = jnp.sum(jnp.where(lane == 0, exv, 0.0))
                        ab = jnp.broadcast_to(a0, (16,))
                        msg[p][r, :] = ab * whr[p][r, :]
                pltpu.async_copy(exc[p], sacc.at[sdst[p]], ssem[p], add=True)
                pltpu.async_copy(msg[p], oacc.at[sdst[p]], ssem[p], add=True)

        wait_idx(NCHK - 1, 1)
        wait_gathers(0)
        wait_scatters(0)
        wait_scatters(1)
        plsc.subcore_barrier()
        pltpu.sync_copy(sacc.at[pl.ds(r0, RPS)], s_hbm.at[c, pl.ds(r0, RPS)])
        pltpu.sync_copy(oacc.at[pl.ds(r0, RPS)], out_hbm.at[c, pl.ds(r0, RPS)])

    return pl.kernel(
        body,
        out_type=(jax.ShapeDtypeStruct((NC, NP, 16), jnp.float32),
                  jax.ShapeDtypeStruct((NC, NP, wf), jnp.float32)),
        mesh=_mesh,
        scratch_types=([pltpu.VMEM((CH,), jnp.int32)] * 6 +
                       [pltpu.VMEM((CH, 16), jnp.float32)] * 4 +
                       [pltpu.VMEM((CH, wf), jnp.float32)] * 2 +
                       [pltpu.VMEM((CH, 16), jnp.float32)] * 2 +
                       [pltpu.VMEM((CH, wf), jnp.float32)] * 2 +
                       [pltpu.VMEM_SHARED((NP, 16), jnp.float32),
                        pltpu.VMEM_SHARED((NP, wf), jnp.float32)] +
                       [pltpu.SemaphoreType.DMA] * 6),
        compiler_params=_SC_PARAMS,
    )


_sc_layer64 = _make_sc_layer(HF, True)
_sc_layer16 = _make_sc_layer(16, False)


def _proj1_body(x_ref, w_ref, wh_ref, ta_ref, tb_ref):
    y = jnp.dot(x_ref[...], w_ref[...], preferred_element_type=jnp.float32)
    wh_ref[...] = y[:, :HF]
    rows = pl.program_id(0) * BLK + lax.broadcasted_iota(jnp.int32, (BLK, 16), 0)
    valid = rows < N
    ta_ref[...] = jnp.where(valid, y[:, HF:HF + 16], BIG)
    tb_ref[...] = jnp.where(valid, y[:, HF + 16:HF + 32], BIG)


_proj1 = pl.pallas_call(
    _proj1_body,
    grid=(NP // BLK,),
    in_specs=[pl.BlockSpec((BLK, D), lambda i: (i, 0)),
              pl.BlockSpec((D, D), lambda i: (0, 0))],
    out_specs=[pl.BlockSpec((BLK, HF), lambda i: (i, 0)),
               pl.BlockSpec((BLK, 16), lambda i: (i, 0)),
               pl.BlockSpec((BLK, 16), lambda i: (i, 0))],
    out_shape=[jax.ShapeDtypeStruct((NP, HF), jnp.float32),
               jax.ShapeDtypeStruct((NP, 16), jnp.float32),
               jax.ShapeDtypeStruct((NP, 16), jnp.float32)],
)


def _proj2_body(oa_ref, ob_ref, sa_ref, sb_ref, w_ref, wh_ref, ta_ref, tb_ref):
    s = sa_ref[...] + sb_ref[...] + 1e-30
    sx = jnp.concatenate(
        [jnp.broadcast_to(s[:, hh:hh + 1], (BLK, 8)) for hh in range(8)],
        axis=1)
    hsum = (oa_ref[...] + ob_ref[...]) / sx
    h = jnp.where(hsum > 0, hsum, jnp.exp(hsum) - 1.0)
    y = jnp.dot(h, w_ref[...], preferred_element_type=jnp.float32)
    col = lax.broadcasted_iota(jnp.int32, (BLK, 16), 1)
    rows = pl.program_id(0) * BLK + lax.broadcasted_iota(jnp.int32, (BLK, 16), 0)
    valid = rows < N
    wh_ref[...] = jnp.where(col < 7, y, 0.0)
    asl = y[:, 7:8]
    adl = y[:, 8:9]
    ta = jnp.where(col == 0, asl, 0.0) + jnp.where(col == 8, adl, 0.0)
    tb = jnp.where(col == 0, adl, 0.0) + jnp.where(col == 8, asl, 0.0)
    ta_ref[...] = jnp.where(valid, ta, BIG)
    tb_ref[...] = jnp.where(valid, tb, BIG)


_proj2 = pl.pallas_call(
    _proj2_body,
    grid=(NP // BLK,),
    in_specs=[pl.BlockSpec((BLK, HF), lambda i: (i, 0)),
              pl.BlockSpec((BLK, HF), lambda i: (i, 0)),
              pl.BlockSpec((BLK, 16), lambda i: (i, 0)),
              pl.BlockSpec((BLK, 16), lambda i: (i, 0)),
              pl.BlockSpec((HF, 16), lambda i: (0, 0))],
    out_specs=[pl.BlockSpec((BLK, 16), lambda i: (i, 0)),
               pl.BlockSpec((BLK, 16), lambda i: (i, 0)),
               pl.BlockSpec((BLK, 16), lambda i: (i, 0))],
    out_shape=[jax.ShapeDtypeStruct((NP, 16), jnp.float32),
               jax.ShapeDtypeStruct((NP, 16), jnp.float32),
               jax.ShapeDtypeStruct((NP, 16), jnp.float32)],
)


def _smax_body(oa_ref, ob_ref, sa_ref, sb_ref, o_ref):
    s = sa_ref[...] + sb_ref[...] + 1e-30
    z = (oa_ref[...] + ob_ref[...]) / s[:, 0:1]
    col = lax.broadcasted_iota(jnp.int32, z.shape, 1)
    zm = jnp.where(col < 7, z, BIG)
    m = jnp.max(zm, axis=1, keepdims=True)
    e = jnp.exp(zm - m)
    o_ref[...] = e / jnp.sum(e, axis=1, keepdims=True)


_smax = pl.pallas_call(
    _smax_body,
    grid=(NP // BLK,),
    in_specs=[pl.BlockSpec((BLK, 16), lambda i: (i, 0)),
              pl.BlockSpec((BLK, 16), lambda i: (i, 0)),
              pl.BlockSpec((BLK, 16), lambda i: (i, 0)),
              pl.BlockSpec((BLK, 16), lambda i: (i, 0))],
    out_specs=pl.BlockSpec((BLK, 16), lambda i: (i, 0)),
    out_shape=jax.ShapeDtypeStruct((NP, 16), jnp.float32),
)


@jax.jit
def kernel(x, edge_index, W1, a_src1, a_dst1, W2, a_src2, a_dst2):
    f32 = jnp.float32
    W1r = jnp.transpose(W1, (1, 0, 2)).reshape(D, HF)
    rows64 = jnp.arange(HF)
    heads = rows64 // 8
    As = jnp.zeros((HF, 8), f32).at[rows64, heads].set(a_src1.reshape(-1))
    Ad = jnp.zeros((HF, 8), f32).at[rows64, heads].set(a_dst1.reshape(-1))
    Wc1 = jnp.concatenate(
        [W1r, W1r @ As, W1r @ Ad, W1r @ Ad, W1r @ As,
         jnp.zeros((D, D - HF - 32), f32)], axis=1)
    W2r = W2[0]
    as2 = a_src2.reshape(-1)
    ad2 = a_dst2.reshape(-1)
    Wc2 = jnp.concatenate(
        [W2r, (W2r @ as2)[:, None], (W2r @ ad2)[:, None],
         jnp.zeros((HF, 7), f32)], axis=1)
    xp = jnp.pad(x, ((0, NP - N), (0, 0)))
    loops = jnp.arange(N, dtype=jnp.int32)
    padi = jnp.full((EPAD - E1,), N, jnp.int32)
    src = jnp.concatenate([edge_index[0], loops, padi])
    dst = jnp.concatenate([edge_index[1], loops, padi])
    zn16 = jnp.zeros((NP, 16), f32)
    zn64 = jnp.zeros((NP, HF), f32)

    wh1, t1a, t1b = _proj1(xp, Wc1)
    s1, o1 = _sc_layer64(t1a, t1b, wh1, src, dst, zn16, zn64)
    wh2, t2a, t2b = _proj2(o1[0], o1[1], s1[0], s1[1], Wc2)
    s2, o2 = _sc_layer16(t2a, t2b, wh2, src, dst, zn16, zn16)
    probs = _smax(o2[0], o2[1], s2[0], s2[1])
    return probs[:N, :7]


# speedup vs baseline: 139.9763x; 1.0629x over previous
import jax
import jax.numpy as jnp
from jax import lax
from jax.experimental import pallas as pl
from jax.experimental.pallas import tpu as pltpu
from jax.experimental.pallas import tpu_sc as plsc

N = 10000
NP = 10240
D = 128
HF = 64
E = 320000
E1 = E + N
NC, NS = 2, 16
NW = NC * NS
CH = 128
NCHK = 82
EPW = NCHK * CH
EPAD = EPW * NW
RPS = NP // NS
BLK = 1024
BIG = -1e30

_mesh = plsc.VectorSubcoreMesh(core_axis_name="c", subcore_axis_name="s",
                               num_cores=NC, num_subcores=NS)


_SC_PARAMS = pltpu.CompilerParams(use_tc_tiling_on_sc=False,
                                  needs_layout_passes=False)


def _make_sc_layer(wf, multi_head):
    nb = wf // 16

    def body(ta_hbm, tb_hbm, wh_hbm, src_hbm, dst_hbm, zn16_hbm, znw_hbm,
             s_hbm, out_hbm,
             src0, src1, dst0, dst1, sdst0, sdst1,
             ra0, ra1, rb0, rb1, whr0, whr1, exc0, exc1, msg0, msg1,
             sacc, oacc, gsem0, gsem1, isem0, isem1, ssem0, ssem1):
        src_c = [src0, src1]
        dst_c = [dst0, dst1]
        sdst = [sdst0, sdst1]
        ra = [ra0, ra1]
        rb = [rb0, rb1]
        whr = [whr0, whr1]
        exc = [exc0, exc1]
        msg = [msg0, msg1]
        gsem = [gsem0, gsem1]
        isem = [isem0, isem1]
        ssem = [ssem0, ssem1]

        c = lax.axis_index("c")
        sid = lax.axis_index("s")
        wid = sid * NC + c
        r0 = sid * RPS
        pltpu.sync_copy(zn16_hbm.at[pl.ds(r0, RPS)], sacc.at[pl.ds(r0, RPS)])
        pltpu.sync_copy(znw_hbm.at[pl.ds(r0, RPS)], oacc.at[pl.ds(r0, RPS)])
        plsc.subcore_barrier()
        base = wid * EPW

        def idx_off(n):
            nn = jnp.minimum(n, NCHK - 1)
            return pl.multiple_of(base + nn * CH, CH)

        def issue_gathers(q):
            pltpu.async_copy(ta_hbm.at[src_c[q]], ra[q], gsem[q])
            pltpu.async_copy(tb_hbm.at[dst_c[q]], rb[q], gsem[q])
            pltpu.async_copy(wh_hbm.at[src_c[q]], whr[q], gsem[q])

        def wait_gathers(q):
            pltpu.make_async_copy(ta_hbm.at[src_c[q]], ra[q], gsem[q]).wait()
            pltpu.make_async_copy(tb_hbm.at[dst_c[q]], rb[q], gsem[q]).wait()
            pltpu.make_async_copy(wh_hbm.at[src_c[q]], whr[q], gsem[q]).wait()

        def issue_idx(n, q, sync=False):
            off = idx_off(n)
            if sync:
                pltpu.sync_copy(src_hbm.at[pl.ds(off, CH)], src_c[q])
                pltpu.sync_copy(dst_hbm.at[pl.ds(off, CH)], dst_c[q])
            else:
                pltpu.async_copy(src_hbm.at[pl.ds(off, CH)], src_c[q], isem[q])
                pltpu.async_copy(dst_hbm.at[pl.ds(off, CH)], dst_c[q], isem[q])

        def wait_idx(n, q):
            off = idx_off(n)
            pltpu.make_async_copy(src_hbm.at[pl.ds(off, CH)], src_c[q], isem[q]).wait()
            pltpu.make_async_copy(dst_hbm.at[pl.ds(off, CH)], dst_c[q], isem[q]).wait()

        def wait_scatters(p):
            pltpu.make_async_copy(exc[p], sacc.at[sdst[p]], ssem[p]).wait()
            pltpu.make_async_copy(msg[p], oacc.at[sdst[p]], ssem[p]).wait()

        issue_idx(0, 0, sync=True)
        issue_idx(1, 1, sync=True)
        issue_gathers(0)

        @pl.loop(0, NCHK, step=2)
        def _pair(i):
            for b in range(2):
                p, q = b, 1 - b
                n = i + b
                @pl.when(n >= 2)
                def _(p=p):
                    wait_scatters(p)
                @pl.when(n >= 1)
                def _(n=n, q=q):
                    wait_idx(n + 1, q)
                issue_gathers(q)
                wait_gathers(p)
                for k in range(CH // 16):
                    sdst[p][pl.ds(16 * k, 16)] = dst_c[p][pl.ds(16 * k, 16)]
                issue_idx(n + 2, p)
                lane = lax.iota(jnp.int32, 16)
                zero16 = lane & 0
                lane_h = lane >> 3
                for r in range(CH):
                    v = ra[p][r, :] + rb[p][r, :]
                    v = jnp.where(v > 0, v, 0.2 * v)
                    exc[p][r, :] = jnp.exp(v)
                if multi_head:
                    for r in range(CH):
                        for j in range(nb):
                            ab = plsc.load_gather(
                                exc[p], [zero16 + r, lane_h + 2 * j])
                            msg[p][r, pl.ds(16 * j, 16)] = (
                                ab * whr[p][r, pl.ds(16 * j, 16)])
                else:
                    for r in range(CH):
                        a0 = jnp.sum(jnp.where(lane == 0, exc[p][r, :], 0.0))
                        ab = jnp.broadcast_to(a0, (16,))
                        msg[p][r, :] = ab * whr[p][r, :]
                pltpu.async_copy(exc[p], sacc.at[sdst[p]], ssem[p], add=True)
                pltpu.async_copy(msg[p], oacc.at[sdst[p]], ssem[p], add=True)

        wait_idx(NCHK - 1, 1)
        wait_gathers(0)
        wait_scatters(0)
        wait_scatters(1)
        plsc.subcore_barrier()
        pltpu.sync_copy(sacc.at[pl.ds(r0, RPS)], s_hbm.at[c, pl.ds(r0, RPS)])
        pltpu.sync_copy(oacc.at[pl.ds(r0, RPS)], out_hbm.at[c, pl.ds(r0, RPS)])

    return pl.kernel(
        body,
        out_type=(jax.ShapeDtypeStruct((NC, NP, 16), jnp.float32),
                  jax.ShapeDtypeStruct((NC, NP, wf), jnp.float32)),
        mesh=_mesh,
        scratch_types=([pltpu.VMEM((CH,), jnp.int32)] * 6 +
                       [pltpu.VMEM((CH, 16), jnp.float32)] * 4 +
                       [pltpu.VMEM((CH, wf), jnp.float32)] * 2 +
                       [pltpu.VMEM((CH, 16), jnp.float32)] * 2 +
                       [pltpu.VMEM((CH, wf), jnp.float32)] * 2 +
                       [pltpu.VMEM_SHARED((NP, 16), jnp.float32),
                        pltpu.VMEM_SHARED((NP, wf), jnp.float32)] +
                       [pltpu.SemaphoreType.DMA] * 6),
        compiler_params=_SC_PARAMS,
    )


_sc_layer64 = _make_sc_layer(HF, True)
_sc_layer16 = _make_sc_layer(16, False)


def _proj1_body(x_ref, w_ref, wh_ref, ta_ref, tb_ref):
    y = jnp.dot(x_ref[...], w_ref[...], preferred_element_type=jnp.float32)
    wh_ref[...] = y[:, :HF]
    rows = pl.program_id(0) * BLK + lax.broadcasted_iota(jnp.int32, (BLK, 16), 0)
    valid = rows < N
    ta_ref[...] = jnp.where(valid, y[:, HF:HF + 16], BIG)
    tb_ref[...] = jnp.where(valid, y[:, HF + 16:HF + 32], BIG)


_proj1 = pl.pallas_call(
    _proj1_body,
    grid=(NP // BLK,),
    in_specs=[pl.BlockSpec((BLK, D), lambda i: (i, 0)),
              pl.BlockSpec((D, D), lambda i: (0, 0))],
    out_specs=[pl.BlockSpec((BLK, HF), lambda i: (i, 0)),
               pl.BlockSpec((BLK, 16), lambda i: (i, 0)),
               pl.BlockSpec((BLK, 16), lambda i: (i, 0))],
    out_shape=[jax.ShapeDtypeStruct((NP, HF), jnp.float32),
               jax.ShapeDtypeStruct((NP, 16), jnp.float32),
               jax.ShapeDtypeStruct((NP, 16), jnp.float32)],
)


def _proj2_body(oa_ref, ob_ref, sa_ref, sb_ref, w_ref, wh_ref, ta_ref, tb_ref):
    s = sa_ref[...] + sb_ref[...] + 1e-30
    sx = jnp.concatenate(
        [jnp.broadcast_to(s[:, hh:hh + 1], (BLK, 8)) for hh in range(8)],
        axis=1)
    hsum = (oa_ref[...] + ob_ref[...]) / sx
    h = jnp.where(hsum > 0, hsum, jnp.exp(hsum) - 1.0)
    y = jnp.dot(h, w_ref[...], preferred_element_type=jnp.float32)
    col = lax.broadcasted_iota(jnp.int32, (BLK, 16), 1)
    rows = pl.program_id(0) * BLK + lax.broadcasted_iota(jnp.int32, (BLK, 16), 0)
    valid = rows < N
    wh_ref[...] = jnp.where(col < 7, y, 0.0)
    asl = y[:, 7:8]
    adl = y[:, 8:9]
    ta = jnp.where(col == 0, asl, 0.0) + jnp.where(col == 8, adl, 0.0)
    tb = jnp.where(col == 0, adl, 0.0) + jnp.where(col == 8, asl, 0.0)
    ta_ref[...] = jnp.where(valid, ta, BIG)
    tb_ref[...] = jnp.where(valid, tb, BIG)


_proj2 = pl.pallas_call(
    _proj2_body,
    grid=(NP // BLK,),
    in_specs=[pl.BlockSpec((BLK, HF), lambda i: (i, 0)),
              pl.BlockSpec((BLK, HF), lambda i: (i, 0)),
              pl.BlockSpec((BLK, 16), lambda i: (i, 0)),
              pl.BlockSpec((BLK, 16), lambda i: (i, 0)),
              pl.BlockSpec((HF, 16), lambda i: (0, 0))],
    out_specs=[pl.BlockSpec((BLK, 16), lambda i: (i, 0)),
               pl.BlockSpec((BLK, 16), lambda i: (i, 0)),
               pl.BlockSpec((BLK, 16), lambda i: (i, 0))],
    out_shape=[jax.ShapeDtypeStruct((NP, 16), jnp.float32),
               jax.ShapeDtypeStruct((NP, 16), jnp.float32),
               jax.ShapeDtypeStruct((NP, 16), jnp.float32)],
)


def _smax_body(oa_ref, ob_ref, sa_ref, sb_ref, o_ref):
    s = sa_ref[...] + sb_ref[...] + 1e-30
    z = (oa_ref[...] + ob_ref[...]) / s[:, 0:1]
    col = lax.broadcasted_iota(jnp.int32, z.shape, 1)
    zm = jnp.where(col < 7, z, BIG)
    m = jnp.max(zm, axis=1, keepdims=True)
    e = jnp.exp(zm - m)
    o_ref[...] = e / jnp.sum(e, axis=1, keepdims=True)


_smax = pl.pallas_call(
    _smax_body,
    grid=(NP // BLK,),
    in_specs=[pl.BlockSpec((BLK, 16), lambda i: (i, 0)),
              pl.BlockSpec((BLK, 16), lambda i: (i, 0)),
              pl.BlockSpec((BLK, 16), lambda i: (i, 0)),
              pl.BlockSpec((BLK, 16), lambda i: (i, 0))],
    out_specs=pl.BlockSpec((BLK, 16), lambda i: (i, 0)),
    out_shape=jax.ShapeDtypeStruct((NP, 16), jnp.float32),
)


@jax.jit
def kernel(x, edge_index, W1, a_src1, a_dst1, W2, a_src2, a_dst2):
    f32 = jnp.float32
    W1r = jnp.transpose(W1, (1, 0, 2)).reshape(D, HF)
    rows64 = jnp.arange(HF)
    heads = rows64 // 8
    As = jnp.zeros((HF, 8), f32).at[rows64, heads].set(a_src1.reshape(-1))
    Ad = jnp.zeros((HF, 8), f32).at[rows64, heads].set(a_dst1.reshape(-1))
    Wc1 = jnp.concatenate(
        [W1r, W1r @ As, W1r @ Ad, W1r @ Ad, W1r @ As,
         jnp.zeros((D, D - HF - 32), f32)], axis=1)
    W2r = W2[0]
    as2 = a_src2.reshape(-1)
    ad2 = a_dst2.reshape(-1)
    Wc2 = jnp.concatenate(
        [W2r, (W2r @ as2)[:, None], (W2r @ ad2)[:, None],
         jnp.zeros((HF, 7), f32)], axis=1)
    xp = jnp.pad(x, ((0, NP - N), (0, 0)))
    loops = jnp.arange(N, dtype=jnp.int32)
    padi = jnp.full((EPAD - E1,), N, jnp.int32)
    src = jnp.concatenate([edge_index[0], loops, padi])
    dst = jnp.concatenate([edge_index[1], loops, padi])
    zn16 = jnp.zeros((NP, 16), f32)
    zn64 = jnp.zeros((NP, HF), f32)

    wh1, t1a, t1b = _proj1(xp, Wc1)
    s1, o1 = _sc_layer64(t1a, t1b, wh1, src, dst, zn16, zn64)
    wh2, t2a, t2b = _proj2(o1[0], o1[1], s1[0], s1[1], Wc2)
    s2, o2 = _sc_layer16(t2a, t2b, wh2, src, dst, zn16, zn16)
    probs = _smax(o2[0], o2[1], s2[0], s2[1])
    return probs[:N, :7]


# speedup vs baseline: 193.0999x; 1.3795x over previous
import jax
import jax.numpy as jnp
from jax import lax
from jax.experimental import pallas as pl
from jax.experimental.pallas import tpu as pltpu
from jax.experimental.pallas import tpu_sc as plsc

N = 10000
NP = 10240
D = 128
HF = 64
E = 320000
E1 = E + N
NC, NS = 2, 16
NW = NC * NS
CH = 128
NCHK = 82
EPW = NCHK * CH
EPAD = EPW * NW
RPS = NP // NS
BLK = 1024
BIG = -1e30

_mesh = plsc.VectorSubcoreMesh(core_axis_name="c", subcore_axis_name="s",
                               num_cores=NC, num_subcores=NS)


_SC_PARAMS = pltpu.CompilerParams(use_tc_tiling_on_sc=False,
                                  needs_layout_passes=False)


def _make_sc_layer(wf, multi_head):
    nb = wf // 16

    def body(ta_hbm, tb_hbm, wh_hbm, src_hbm, dst_hbm, zn16_hbm, znw_hbm,
             s_hbm, out_hbm,
             src0, src1, dst0, dst1, sdst0, sdst1,
             ra0, ra1, rb0, rb1, whr0, whr1, exc0, exc1, msg0, msg1,
             sacc, oacc, gsem0, gsem1, isem0, isem1, ssem0, ssem1):
        src_c = [src0, src1]
        dst_c = [dst0, dst1]
        sdst = [sdst0, sdst1]
        ra = [ra0, ra1]
        rb = [rb0, rb1]
        whr = [whr0, whr1]
        exc = [exc0, exc1]
        msg = [msg0, msg1]
        gsem = [gsem0, gsem1]
        isem = [isem0, isem1]
        ssem = [ssem0, ssem1]

        c = lax.axis_index("c")
        sid = lax.axis_index("s")
        wid = sid * NC + c
        r0 = sid * RPS
        pltpu.sync_copy(zn16_hbm.at[pl.ds(r0, RPS)], sacc.at[pl.ds(r0, RPS)])
        pltpu.sync_copy(znw_hbm.at[pl.ds(r0, RPS)], oacc.at[pl.ds(r0, RPS)])
        plsc.subcore_barrier()
        base = wid * EPW

        def idx_off(n):
            nn = jnp.minimum(n, NCHK - 1)
            return pl.multiple_of(base + nn * CH, CH)

        def issue_gathers(q):
            pltpu.async_copy(ta_hbm.at[src_c[q]], ra[q], gsem[q])
            pltpu.async_copy(tb_hbm.at[dst_c[q]], rb[q], gsem[q])
            pltpu.async_copy(wh_hbm.at[src_c[q]], whr[q], gsem[q])

        def wait_gathers(q):
            pltpu.make_async_copy(ta_hbm.at[src_c[q]], ra[q], gsem[q]).wait()
            pltpu.make_async_copy(tb_hbm.at[dst_c[q]], rb[q], gsem[q]).wait()
            pltpu.make_async_copy(wh_hbm.at[src_c[q]], whr[q], gsem[q]).wait()

        def issue_idx(n, q, sync=False):
            off = idx_off(n)
            if sync:
                pltpu.sync_copy(src_hbm.at[pl.ds(off, CH)], src_c[q])
                pltpu.sync_copy(dst_hbm.at[pl.ds(off, CH)], dst_c[q])
            else:
                pltpu.async_copy(src_hbm.at[pl.ds(off, CH)], src_c[q], isem[q])
                pltpu.async_copy(dst_hbm.at[pl.ds(off, CH)], dst_c[q], isem[q])

        def wait_idx(n, q):
            off = idx_off(n)
            pltpu.make_async_copy(src_hbm.at[pl.ds(off, CH)], src_c[q], isem[q]).wait()
            pltpu.make_async_copy(dst_hbm.at[pl.ds(off, CH)], dst_c[q], isem[q]).wait()

        def wait_scatters(p):
            pltpu.make_async_copy(exc[p], sacc.at[sdst[p]], ssem[p]).wait()
            pltpu.make_async_copy(msg[p], oacc.at[sdst[p]], ssem[p]).wait()

        issue_idx(0, 0, sync=True)
        issue_idx(1, 1, sync=True)
        issue_gathers(0)

        @pl.loop(0, NCHK, step=2)
        def _pair(i):
            for b in range(2):
                p, q = b, 1 - b
                n = i + b
                @pl.when(n >= 2)
                def _(p=p):
                    wait_scatters(p)
                @pl.when(n >= 1)
                def _(n=n, q=q):
                    wait_idx(n + 1, q)
                issue_gathers(q)
                wait_gathers(p)
                for k in range(CH // 16):
                    sdst[p][pl.ds(16 * k, 16)] = dst_c[p][pl.ds(16 * k, 16)]
                issue_idx(n + 2, p)
                lane = lax.iota(jnp.int32, 16)
                zero16 = lane & 0
                lane_h = lane >> 3
                for r in range(CH):
                    v = ra[p][r, :] + rb[p][r, :]
                    v = jnp.where(v > 0, v, 0.2 * v)
                    exc[p][r, :] = jnp.exp(v)
                if multi_head:
                    for r in range(CH):
                        ab = plsc.load_gather(exc[p], [zero16 + r, lane & 7])
                        for j in range(nb):
                            msg[p][r, pl.ds(16 * j, 16)] = (
                                ab * whr[p][r, pl.ds(16 * j, 16)])
                else:
                    for r in range(CH):
                        a0 = jnp.sum(jnp.where(lane == 0, exc[p][r, :], 0.0))
                        ab = jnp.broadcast_to(a0, (16,))
                        msg[p][r, :] = ab * whr[p][r, :]
                pltpu.async_copy(exc[p], sacc.at[sdst[p]], ssem[p], add=True)
                pltpu.async_copy(msg[p], oacc.at[sdst[p]], ssem[p], add=True)

        wait_idx(NCHK - 1, 1)
        wait_gathers(0)
        wait_scatters(0)
        wait_scatters(1)
        plsc.subcore_barrier()
        pltpu.sync_copy(sacc.at[pl.ds(r0, RPS)], s_hbm.at[c, pl.ds(r0, RPS)])
        pltpu.sync_copy(oacc.at[pl.ds(r0, RPS)], out_hbm.at[c, pl.ds(r0, RPS)])

    return pl.kernel(
        body,
        out_type=(jax.ShapeDtypeStruct((NC, NP, 16), jnp.float32),
                  jax.ShapeDtypeStruct((NC, NP, wf), jnp.float32)),
        mesh=_mesh,
        scratch_types=([pltpu.VMEM((CH,), jnp.int32)] * 6 +
                       [pltpu.VMEM((CH, 16), jnp.float32)] * 4 +
                       [pltpu.VMEM((CH, wf), jnp.float32)] * 2 +
                       [pltpu.VMEM((CH, 16), jnp.float32)] * 2 +
                       [pltpu.VMEM((CH, wf), jnp.float32)] * 2 +
                       [pltpu.VMEM_SHARED((NP, 16), jnp.float32),
                        pltpu.VMEM_SHARED((NP, wf), jnp.float32)] +
                       [pltpu.SemaphoreType.DMA] * 6),
        compiler_params=_SC_PARAMS,
    )


_sc_layer64 = _make_sc_layer(HF, True)
_sc_layer16 = _make_sc_layer(16, False)


def _proj1_body(x_ref, w_ref, wh_ref, ta_ref, tb_ref):
    y = jnp.dot(x_ref[...], w_ref[...], preferred_element_type=jnp.float32)
    wh_ref[...] = y[:, :HF]
    rows = pl.program_id(0) * BLK + lax.broadcasted_iota(jnp.int32, (BLK, 16), 0)
    valid = rows < N
    ta_ref[...] = jnp.where(valid, y[:, HF:HF + 16], BIG)
    tb_ref[...] = jnp.where(valid, y[:, HF + 16:HF + 32], BIG)


_proj1 = pl.pallas_call(
    _proj1_body,
    grid=(NP // BLK,),
    in_specs=[pl.BlockSpec((BLK, D), lambda i: (i, 0)),
              pl.BlockSpec((D, D), lambda i: (0, 0))],
    out_specs=[pl.BlockSpec((BLK, HF), lambda i: (i, 0)),
               pl.BlockSpec((BLK, 16), lambda i: (i, 0)),
               pl.BlockSpec((BLK, 16), lambda i: (i, 0))],
    out_shape=[jax.ShapeDtypeStruct((NP, HF), jnp.float32),
               jax.ShapeDtypeStruct((NP, 16), jnp.float32),
               jax.ShapeDtypeStruct((NP, 16), jnp.float32)],
)


def _proj2_body(oa_ref, ob_ref, sa_ref, sb_ref, w_ref, wh_ref, ta_ref, tb_ref):
    s = sa_ref[...] + sb_ref[...] + 1e-30
    sx = jnp.concatenate([s[:, :8]] * 8, axis=1)
    hsum = (oa_ref[...] + ob_ref[...]) / sx
    h = jnp.where(hsum > 0, hsum, jnp.exp(hsum) - 1.0)
    y = jnp.dot(h, w_ref[...], preferred_element_type=jnp.float32)
    col = lax.broadcasted_iota(jnp.int32, (BLK, 16), 1)
    rows = pl.program_id(0) * BLK + lax.broadcasted_iota(jnp.int32, (BLK, 16), 0)
    valid = rows < N
    wh_ref[...] = jnp.where(col < 7, y, 0.0)
    asl = y[:, 7:8]
    adl = y[:, 8:9]
    ta = jnp.where(col == 0, asl, 0.0) + jnp.where(col == 8, adl, 0.0)
    tb = jnp.where(col == 0, adl, 0.0) + jnp.where(col == 8, asl, 0.0)
    ta_ref[...] = jnp.where(valid, ta, BIG)
    tb_ref[...] = jnp.where(valid, tb, BIG)


_proj2 = pl.pallas_call(
    _proj2_body,
    grid=(NP // BLK,),
    in_specs=[pl.BlockSpec((BLK, HF), lambda i: (i, 0)),
              pl.BlockSpec((BLK, HF), lambda i: (i, 0)),
              pl.BlockSpec((BLK, 16), lambda i: (i, 0)),
              pl.BlockSpec((BLK, 16), lambda i: (i, 0)),
              pl.BlockSpec((HF, 16), lambda i: (0, 0))],
    out_specs=[pl.BlockSpec((BLK, 16), lambda i: (i, 0)),
               pl.BlockSpec((BLK, 16), lambda i: (i, 0)),
               pl.BlockSpec((BLK, 16), lambda i: (i, 0))],
    out_shape=[jax.ShapeDtypeStruct((NP, 16), jnp.float32),
               jax.ShapeDtypeStruct((NP, 16), jnp.float32),
               jax.ShapeDtypeStruct((NP, 16), jnp.float32)],
)


def _smax_body(oa_ref, ob_ref, sa_ref, sb_ref, o_ref):
    s = sa_ref[...] + sb_ref[...] + 1e-30
    z = (oa_ref[...] + ob_ref[...]) / s[:, 0:1]
    col = lax.broadcasted_iota(jnp.int32, z.shape, 1)
    zm = jnp.where(col < 7, z, BIG)
    m = jnp.max(zm, axis=1, keepdims=True)
    e = jnp.exp(zm - m)
    o_ref[...] = e / jnp.sum(e, axis=1, keepdims=True)


_smax = pl.pallas_call(
    _smax_body,
    grid=(NP // BLK,),
    in_specs=[pl.BlockSpec((BLK, 16), lambda i: (i, 0)),
              pl.BlockSpec((BLK, 16), lambda i: (i, 0)),
              pl.BlockSpec((BLK, 16), lambda i: (i, 0)),
              pl.BlockSpec((BLK, 16), lambda i: (i, 0))],
    out_specs=pl.BlockSpec((BLK, 16), lambda i: (i, 0)),
    out_shape=jax.ShapeDtypeStruct((NP, 16), jnp.float32),
)


@jax.jit
def kernel(x, edge_index, W1, a_src1, a_dst1, W2, a_src2, a_dst2):
    f32 = jnp.float32
    W1r = jnp.transpose(W1, (1, 0, 2)).reshape(D, HF)
    rows64 = jnp.arange(HF)
    heads = rows64 // 8
    As = jnp.zeros((HF, 8), f32).at[rows64, heads].set(a_src1.reshape(-1))
    Ad = jnp.zeros((HF, 8), f32).at[rows64, heads].set(a_dst1.reshape(-1))
    perm = (jnp.arange(HF) % 8) * 8 + jnp.arange(HF) // 8
    Wc1 = jnp.concatenate(
        [W1r[:, perm], W1r @ As, W1r @ Ad, W1r @ Ad, W1r @ As,
         jnp.zeros((D, D - HF - 32), f32)], axis=1)
    W2r = W2[0]
    as2 = a_src2.reshape(-1)
    ad2 = a_dst2.reshape(-1)
    Wc2 = jnp.concatenate(
        [W2r, (W2r @ as2)[:, None], (W2r @ ad2)[:, None],
         jnp.zeros((HF, 7), f32)], axis=1)[perm, :]
    xp = jnp.pad(x, ((0, NP - N), (0, 0)))
    loops = jnp.arange(N, dtype=jnp.int32)
    padi = jnp.full((EPAD - E1,), N, jnp.int32)
    src = jnp.concatenate([edge_index[0], loops, padi])
    dst = jnp.concatenate([edge_index[1], loops, padi])
    zn16 = jnp.zeros((NP, 16), f32)
    zn64 = jnp.zeros((NP, HF), f32)

    wh1, t1a, t1b = _proj1(xp, Wc1)
    s1, o1 = _sc_layer64(t1a, t1b, wh1, src, dst, zn16, zn64)
    wh2, t2a, t2b = _proj2(o1[0], o1[1], s1[0], s1[1], Wc2)
    s2, o2 = _sc_layer16(t2a, t2b, wh2, src, dst, zn16, zn16)
    probs = _smax(o2[0], o2[1], s2[0], s2[1])
    return probs[:N, :7]


# speedup vs baseline: 198.2911x; 1.0269x over previous
import jax
import jax.numpy as jnp
from jax import lax
from jax.experimental import pallas as pl
from jax.experimental.pallas import tpu as pltpu
from jax.experimental.pallas import tpu_sc as plsc

N = 10000
NP = 10240
D = 128
HF = 64
E = 320000
E1 = E + N
NC, NS = 2, 16
NW = NC * NS
CH = 128
NCHK = 82
EPW = NCHK * CH
EPAD = EPW * NW
RPS = NP // NS
BLK = 1024
BIG = -1e30

_mesh = plsc.VectorSubcoreMesh(core_axis_name="c", subcore_axis_name="s",
                               num_cores=NC, num_subcores=NS)


_SC_PARAMS = pltpu.CompilerParams(use_tc_tiling_on_sc=False,
                                  needs_layout_passes=False)


def _make_sc_layer(wf, multi_head):
    nb = wf // 16

    def body(ta_hbm, tb_hbm, wh_hbm, src_hbm, dst_hbm, zn16_hbm, znw_hbm,
             s_hbm, out_hbm,
             src0, src1, dst0, dst1, sdst0, sdst1,
             ra0, ra1, rb0, rb1, whr0, whr1, exc0, exc1, msg0, msg1,
             sacc, oacc, gsem0, gsem1, isem0, isem1, ssem0, ssem1):
        src_c = [src0, src1]
        dst_c = [dst0, dst1]
        sdst = [sdst0, sdst1]
        ra = [ra0, ra1]
        rb = [rb0, rb1]
        whr = [whr0, whr1]
        exc = [exc0, exc1]
        msg = [msg0, msg1]
        gsem = [gsem0, gsem1]
        isem = [isem0, isem1]
        ssem = [ssem0, ssem1]

        c = lax.axis_index("c")
        sid = lax.axis_index("s")
        wid = sid * NC + c
        r0 = sid * RPS
        pltpu.sync_copy(zn16_hbm.at[pl.ds(r0, RPS)], sacc.at[pl.ds(r0, RPS)])
        pltpu.sync_copy(znw_hbm.at[pl.ds(r0, RPS)], oacc.at[pl.ds(r0, RPS)])
        plsc.subcore_barrier()
        base = wid * EPW

        def idx_off(n):
            nn = jnp.minimum(n, NCHK - 1)
            return pl.multiple_of(base + nn * CH, CH)

        def issue_gathers(q):
            pltpu.async_copy(ta_hbm.at[src_c[q]], ra[q], gsem[q])
            pltpu.async_copy(tb_hbm.at[dst_c[q]], rb[q], gsem[q])
            pltpu.async_copy(wh_hbm.at[src_c[q]], whr[q], gsem[q])

        def wait_gathers(q):
            pltpu.make_async_copy(ta_hbm.at[src_c[q]], ra[q], gsem[q]).wait()
            pltpu.make_async_copy(tb_hbm.at[dst_c[q]], rb[q], gsem[q]).wait()
            pltpu.make_async_copy(wh_hbm.at[src_c[q]], whr[q], gsem[q]).wait()

        def issue_idx(n, q, sync=False):
            off = idx_off(n)
            if sync:
                pltpu.sync_copy(src_hbm.at[pl.ds(off, CH)], src_c[q])
                pltpu.sync_copy(dst_hbm.at[pl.ds(off, CH)], dst_c[q])
            else:
                pltpu.async_copy(src_hbm.at[pl.ds(off, CH)], src_c[q], isem[q])
                pltpu.async_copy(dst_hbm.at[pl.ds(off, CH)], dst_c[q], isem[q])

        def wait_idx(n, q):
            off = idx_off(n)
            pltpu.make_async_copy(src_hbm.at[pl.ds(off, CH)], src_c[q], isem[q]).wait()
            pltpu.make_async_copy(dst_hbm.at[pl.ds(off, CH)], dst_c[q], isem[q]).wait()

        def wait_scatters(p):
            pltpu.make_async_copy(exc[p], sacc.at[sdst[p]], ssem[p]).wait()
            pltpu.make_async_copy(msg[p], oacc.at[sdst[p]], ssem[p]).wait()

        issue_idx(0, 0, sync=True)
        issue_idx(1, 1, sync=True)
        issue_gathers(0)

        @pl.loop(0, NCHK, step=2)
        def _pair(i):
            for b in range(2):
                p, q = b, 1 - b
                n = i + b
                @pl.when(n >= 2)
                def _(p=p):
                    wait_scatters(p)
                @pl.when(n >= 1)
                def _(n=n, q=q):
                    wait_idx(n + 1, q)
                issue_gathers(q)
                wait_gathers(p)
                for k in range(CH // 16):
                    sdst[p][pl.ds(16 * k, 16)] = dst_c[p][pl.ds(16 * k, 16)]
                issue_idx(n + 2, p)
                lane = lax.iota(jnp.int32, 16)
                zero16 = lane & 0
                for r in range(CH):
                    v = ra[p][r, :] + rb[p][r, :]
                    v = jnp.where(v > 0, v, 0.2 * v)
                    exc[p][r, :] = jnp.exp(v)
                if multi_head:
                    for r in range(CH):
                        ra[p][r, :] = plsc.load_gather(
                            exc[p], [zero16 + r, lane & 7])
                    for r in range(CH):
                        ab = ra[p][r, :]
                        for j in range(nb):
                            msg[p][r, pl.ds(16 * j, 16)] = (
                                ab * whr[p][r, pl.ds(16 * j, 16)])
                else:
                    for r in range(CH):
                        a0 = jnp.sum(jnp.where(lane == 0, exc[p][r, :], 0.0))
                        ab = jnp.broadcast_to(a0, (16,))
                        msg[p][r, :] = ab * whr[p][r, :]
                pltpu.async_copy(exc[p], sacc.at[sdst[p]], ssem[p], add=True)
                pltpu.async_copy(msg[p], oacc.at[sdst[p]], ssem[p], add=True)

        wait_idx(NCHK - 1, 1)
        wait_gathers(0)
        wait_scatters(0)
        wait_scatters(1)
        plsc.subcore_barrier()
        pltpu.sync_copy(sacc.at[pl.ds(r0, RPS)], s_hbm.at[c, pl.ds(r0, RPS)])
        pltpu.sync_copy(oacc.at[pl.ds(r0, RPS)], out_hbm.at[c, pl.ds(r0, RPS)])

    return pl.kernel(
        body,
        out_type=(jax.ShapeDtypeStruct((NC, NP, 16), jnp.float32),
                  jax.ShapeDtypeStruct((NC, NP, wf), jnp.float32)),
        mesh=_mesh,
        scratch_types=([pltpu.VMEM((CH,), jnp.int32)] * 6 +
                       [pltpu.VMEM((CH, 16), jnp.float32)] * 4 +
                       [pltpu.VMEM((CH, wf), jnp.float32)] * 2 +
                       [pltpu.VMEM((CH, 16), jnp.float32)] * 2 +
                       [pltpu.VMEM((CH, wf), jnp.float32)] * 2 +
                       [pltpu.VMEM_SHARED((NP, 16), jnp.float32),
                        pltpu.VMEM_SHARED((NP, wf), jnp.float32)] +
                       [pltpu.SemaphoreType.DMA] * 6),
        compiler_params=_SC_PARAMS,
    )


_sc_layer64 = _make_sc_layer(HF, True)
_sc_layer16 = _make_sc_layer(16, False)


def _proj1_body(x_ref, w_ref, wh_ref, ta_ref, tb_ref):
    y = jnp.dot(x_ref[...], w_ref[...], preferred_element_type=jnp.float32)
    wh_ref[...] = y[:, :HF]
    rows = pl.program_id(0) * BLK + lax.broadcasted_iota(jnp.int32, (BLK, 16), 0)
    valid = rows < N
    ta_ref[...] = jnp.where(valid, y[:, HF:HF + 16], BIG)
    tb_ref[...] = jnp.where(valid, y[:, HF + 16:HF + 32], BIG)


_proj1 = pl.pallas_call(
    _proj1_body,
    grid=(NP // BLK,),
    in_specs=[pl.BlockSpec((BLK, D), lambda i: (i, 0)),
              pl.BlockSpec((D, D), lambda i: (0, 0))],
    out_specs=[pl.BlockSpec((BLK, HF), lambda i: (i, 0)),
               pl.BlockSpec((BLK, 16), lambda i: (i, 0)),
               pl.BlockSpec((BLK, 16), lambda i: (i, 0))],
    out_shape=[jax.ShapeDtypeStruct((NP, HF), jnp.float32),
               jax.ShapeDtypeStruct((NP, 16), jnp.float32),
               jax.ShapeDtypeStruct((NP, 16), jnp.float32)],
)


def _proj2_body(oa_ref, ob_ref, sa_ref, sb_ref, w_ref, wh_ref, ta_ref, tb_ref):
    s = sa_ref[...] + sb_ref[...] + 1e-30
    sx = jnp.concatenate([s[:, :8]] * 8, axis=1)
    hsum = (oa_ref[...] + ob_ref[...]) / sx
    h = jnp.where(hsum > 0, hsum, jnp.exp(hsum) - 1.0)
    y = jnp.dot(h, w_ref[...], preferred_element_type=jnp.float32)
    col = lax.broadcasted_iota(jnp.int32, (BLK, 16), 1)
    rows = pl.program_id(0) * BLK + lax.broadcasted_iota(jnp.int32, (BLK, 16), 0)
    valid = rows < N
    wh_ref[...] = jnp.where(col < 7, y, 0.0)
    asl = y[:, 7:8]
    adl = y[:, 8:9]
    ta = jnp.where(col == 0, asl, 0.0) + jnp.where(col == 8, adl, 0.0)
    tb = jnp.where(col == 0, adl, 0.0) + jnp.where(col == 8, asl, 0.0)
    ta_ref[...] = jnp.where(valid, ta, BIG)
    tb_ref[...] = jnp.where(valid, tb, BIG)


_proj2 = pl.pallas_call(
    _proj2_body,
    grid=(NP // BLK,),
    in_specs=[pl.BlockSpec((BLK, HF), lambda i: (i, 0)),
              pl.BlockSpec((BLK, HF), lambda i: (i, 0)),
              pl.BlockSpec((BLK, 16), lambda i: (i, 0)),
              pl.BlockSpec((BLK, 16), lambda i: (i, 0)),
              pl.BlockSpec((HF, 16), lambda i: (0, 0))],
    out_specs=[pl.BlockSpec((BLK, 16), lambda i: (i, 0)),
               pl.BlockSpec((BLK, 16), lambda i: (i, 0)),
               pl.BlockSpec((BLK, 16), lambda i: (i, 0))],
    out_shape=[jax.ShapeDtypeStruct((NP, 16), jnp.float32),
               jax.ShapeDtypeStruct((NP, 16), jnp.float32),
               jax.ShapeDtypeStruct((NP, 16), jnp.float32)],
)


def _smax_body(oa_ref, ob_ref, sa_ref, sb_ref, o_ref):
    s = sa_ref[...] + sb_ref[...] + 1e-30
    z = (oa_ref[...] + ob_ref[...]) / s[:, 0:1]
    col = lax.broadcasted_iota(jnp.int32, z.shape, 1)
    zm = jnp.where(col < 7, z, BIG)
    m = jnp.max(zm, axis=1, keepdims=True)
    e = jnp.exp(zm - m)
    o_ref[...] = e / jnp.sum(e, axis=1, keepdims=True)


_smax = pl.pallas_call(
    _smax_body,
    grid=(NP // BLK,),
    in_specs=[pl.BlockSpec((BLK, 16), lambda i: (i, 0)),
              pl.BlockSpec((BLK, 16), lambda i: (i, 0)),
              pl.BlockSpec((BLK, 16), lambda i: (i, 0)),
              pl.BlockSpec((BLK, 16), lambda i: (i, 0))],
    out_specs=pl.BlockSpec((BLK, 16), lambda i: (i, 0)),
    out_shape=jax.ShapeDtypeStruct((NP, 16), jnp.float32),
)


@jax.jit
def kernel(x, edge_index, W1, a_src1, a_dst1, W2, a_src2, a_dst2):
    f32 = jnp.float32
    W1r = jnp.transpose(W1, (1, 0, 2)).reshape(D, HF)
    rows64 = jnp.arange(HF)
    heads = rows64 // 8
    As = jnp.zeros((HF, 8), f32).at[rows64, heads].set(a_src1.reshape(-1))
    Ad = jnp.zeros((HF, 8), f32).at[rows64, heads].set(a_dst1.reshape(-1))
    perm = (jnp.arange(HF) % 8) * 8 + jnp.arange(HF) // 8
    Wc1 = jnp.concatenate(
        [W1r[:, perm], W1r @ As, W1r @ Ad, W1r @ Ad, W1r @ As,
         jnp.zeros((D, D - HF - 32), f32)], axis=1)
    W2r = W2[0]
    as2 = a_src2.reshape(-1)
    ad2 = a_dst2.reshape(-1)
    Wc2 = jnp.concatenate(
        [W2r, (W2r @ as2)[:, None], (W2r @ ad2)[:, None],
         jnp.zeros((HF, 7), f32)], axis=1)[perm, :]
    xp = jnp.pad(x, ((0, NP - N), (0, 0)))
    loops = jnp.arange(N, dtype=jnp.int32)
    padi = jnp.full((EPAD - E1,), N, jnp.int32)
    src = jnp.concatenate([edge_index[0], loops, padi])
    dst = jnp.concatenate([edge_index[1], loops, padi])
    zn16 = jnp.zeros((NP, 16), f32)
    zn64 = jnp.zeros((NP, HF), f32)

    wh1, t1a, t1b = _proj1(xp, Wc1)
    s1, o1 = _sc_layer64(t1a, t1b, wh1, src, dst, zn16, zn64)
    wh2, t2a, t2b = _proj2(o1[0], o1[1], s1[0], s1[1], Wc2)
    s2, o2 = _sc_layer16(t2a, t2b, wh2, src, dst, zn16, zn16)
    probs = _smax(o2[0], o2[1], s2[0], s2[1])
    return probs[:N, :7]
